# baseline (speedup 1.0000x reference)
import functools

import jax
import jax.numpy as jnp
import numpy as np
from jax import lax
from jax.experimental import pallas as pl
from jax.experimental.pallas import tpu as pltpu
from jax.experimental.pallas import tpu_sc as plsc


def _square_distance(src, dst):
    return jnp.sum(src ** 2, -1)[:, :, None] + jnp.sum(dst ** 2, -1)[:, None, :] - 2.0 * jnp.einsum('bnc,bmc->bnm', src, dst)


def _index_points(points, idx):
    B = points.shape[0]
    batch = jnp.arange(B).reshape((B,) + (1,) * (idx.ndim - 1))
    return points[batch, idx]


def _fps_kernel(npoint, xyz_ref, cent_ref):
    B, _, N = xyz_ref.shape
    x = xyz_ref[:, 0, :]
    y = xyz_ref[:, 1, :]
    z = xyz_ref[:, 2, :]
    zn = (x * 0.0).astype(jnp.int32)
    iota_n = jax.lax.broadcasted_iota(jnp.int32, (B, N), 1) + zn

    def body(i, state):
        distance, farthest = state
        sel = iota_n == farthest
        cx = jnp.sum(jnp.where(sel, x, 0.0), axis=1, keepdims=True)
        cy = jnp.sum(jnp.where(sel, y, 0.0), axis=1, keepdims=True)
        cz = jnp.sum(jnp.where(sel, z, 0.0), axis=1, keepdims=True)
        row = jnp.concatenate([farthest.astype(jnp.float32), cx, cy, cz],
                              axis=1)
        cent_ref[pl.ds(i, 1)] = row.reshape(1, B, 4)
        d = (x - cx) ** 2 + (y - cy) ** 2 + (z - cz) ** 2
        distance = jnp.minimum(distance, d)
        maxv = jnp.max(distance, axis=1, keepdims=True)
        farthest = jnp.min(jnp.where(distance == maxv, iota_n, N), axis=1,
                           keepdims=True)
        return distance, farthest

    distance = x * 0.0 + 1e10
    farthest = jnp.min(iota_n, axis=1, keepdims=True) * 0
    jax.lax.fori_loop(0, npoint, body, (distance, farthest))


def _fps(xyz_bcn, npoint):
    B, _, N = xyz_bcn.shape
    cent = pl.pallas_call(
        functools.partial(_fps_kernel, npoint),
        out_shape=jax.ShapeDtypeStruct((npoint, B, 4), jnp.float32),
    )(xyz_bcn)
    return jnp.transpose(cent[:, :, 1:4], (1, 0, 2))


def _query_ball(radius, nsample, xyz, new_xyz):
    B, N, C = xyz.shape
    S = new_xyz.shape[1]
    sqrdists = _square_distance(new_xyz, xyz)
    group_idx = jnp.broadcast_to(jnp.arange(N, dtype=jnp.int32), (B, S, N))
    group_idx = jnp.where(sqrdists > radius ** 2, jnp.int32(N), group_idx)
    group_idx = jnp.sort(group_idx, axis=-1)[:, :, :nsample]
    group_first = jnp.broadcast_to(group_idx[:, :, :1], group_idx.shape)
    group_idx = jnp.where(group_idx == N, group_first, group_idx)
    return jnp.clip(group_idx, 0, N - 1)


def _conv_bn_relu(x, p, eps=1e-5):
    shape = (1, -1) + (1,) * (x.ndim - 2)
    y = jnp.einsum('oi,bi...->bo...', p['W'], x) + p['b'].reshape(shape)
    y = (y - p['mean'].reshape(shape)) / jnp.sqrt(p['var'].reshape(shape) + eps) * p['gamma'].reshape(shape) + p['beta'].reshape(shape)
    return jax.nn.relu(y)


def _set_abstraction(xyz, points, npoint, radius, nsample, layers):
    xyz_t = jnp.transpose(xyz, (0, 2, 1))
    points_t = jnp.transpose(points, (0, 2, 1)) if points is not None else None
    fps_idx = _fps(xyz_t, npoint)
    new_xyz = _index_points(xyz_t, fps_idx)
    idx = _query_ball(radius, nsample, xyz_t, new_xyz)
    grouped_xyz = _index_points(xyz_t, idx) - new_xyz[:, :, None, :]
    if points_t is not None:
        grouped_points = _index_points(points_t, idx)
        new_points = jnp.concatenate([grouped_xyz, grouped_points], axis=-1)
    else:
        new_points = grouped_xyz
    new_points = jnp.transpose(new_points, (0, 3, 2, 1))
    for p in layers:
        new_points = _conv_bn_relu(new_points, p)
    new_points = jnp.max(new_points, axis=2)
    return jnp.transpose(new_xyz, (0, 2, 1)), new_points


def _feature_propagation(xyz1, xyz2, points1, points2, layers):
    xyz1_t = jnp.transpose(xyz1, (0, 2, 1))
    xyz2_t = jnp.transpose(xyz2, (0, 2, 1))
    points2_t = jnp.transpose(points2, (0, 2, 1))
    B, N, _ = xyz1_t.shape
    dists = _square_distance(xyz1_t, xyz2_t)
    idx = jnp.argsort(dists, axis=-1)[:, :, :3]
    d = jnp.take_along_axis(dists, idx, axis=-1)
    dist_recip = 1.0 / (d + 1e-8)
    norm = jnp.sum(dist_recip, axis=2, keepdims=True)
    weight = dist_recip / norm
    interpolated = jnp.sum(_index_points(points2_t, idx) * weight[..., None], axis=2)
    if points1 is not None:
        points1_t = jnp.transpose(points1, (0, 2, 1))
        new_points = jnp.concatenate([points1_t, interpolated], axis=-1)
    else:
        new_points = interpolated
    new_points = jnp.transpose(new_points, (0, 2, 1))
    for p in layers:
        new_points = _conv_bn_relu(new_points, p)
    return new_points


def _sa_dist_kernel(x_ref, nx_ref, d_ref):
    qx = x_ref[0]
    nxr = nx_ref[0]
    pn = jnp.sum(qx * qx, axis=0, keepdims=True)
    sn = jnp.sum(nxr * nxr, axis=1, keepdims=True)
    dot = jax.lax.dot_general(nxr, qx, (((1,), (0,)), ((), ())),
                              preferred_element_type=jnp.float32)
    d_ref[0] = (sn + pn) - 2.0 * dot


def _sa_dist(xyz_bcn, nxyz_rows):
    B, _, N = xyz_bcn.shape
    S = nxyz_rows.shape[1]
    return pl.pallas_call(
        _sa_dist_kernel,
        grid=(B,),
        in_specs=[pl.BlockSpec((1, 3, N), lambda i: (i, 0, 0)),
                  pl.BlockSpec((1, S, 3), lambda i: (i, 0, 0))],
        out_specs=pl.BlockSpec((1, S, N), lambda i: (i, 0, 0)),
        out_shape=jax.ShapeDtypeStruct((B, S, N), jnp.float32),
    )(xyz_bcn, nxyz_rows)


def _select_gather_jnp(D, r2, nsample, tab):
    B, S, N = D.shape
    gi = jnp.broadcast_to(jnp.arange(N, dtype=jnp.int32), (B, S, N))
    gi = jnp.where(D > r2, jnp.int32(N), gi)
    gi = jnp.sort(gi, axis=-1)[:, :, :nsample]
    first = jnp.broadcast_to(gi[:, :, :1], gi.shape)
    gi = jnp.where(gi == N, first, gi)
    gi = jnp.clip(gi, 0, N - 1)
    bidx = jnp.arange(B).reshape(B, 1, 1)
    g = tab[bidx, :, gi]
    return g.reshape(B, S, -1)


def _sa_mlp_kernel(nlayers, S, ns, x_ref, nx_ref, *rest):
    wb = rest[:6 * nlayers]
    out_ref = rest[6 * nlayers]
    F = x_ref[0]
    NX = nx_ref[0]
    rel = F[:, 0:3] - NX
    pts = F if F.shape[1] == 3 else F[:, 3:]
    X = jnp.concatenate([rel, pts], axis=1)
    for l in range(nlayers):
        X = _layer_fwd(X, wb[6 * l:6 * l + 6])
    C = X.shape[1]
    out_ref[0] = jnp.max(X.reshape(S, ns, C), axis=1)


def _sa_mlp(feats, nxyz_rows, layers, nsample):
    B, SK, Cf = feats.shape
    S = SK // nsample
    nxe = jnp.repeat(nxyz_rows, nsample, axis=1)
    wbs = []
    for p in layers:
        wbs += _layer_args(p)
    Cout = wbs[-6].shape[1]
    specs = [pl.BlockSpec((1, SK, Cf), lambda i: (i, 0, 0)),
             pl.BlockSpec((1, SK, 3), lambda i: (i, 0, 0))]
    for a in wbs:
        specs.append(pl.BlockSpec(a.shape, lambda i: (0,) * a.ndim))
    return pl.pallas_call(
        functools.partial(_sa_mlp_kernel, len(layers), S, nsample),
        grid=(B,),
        in_specs=specs,
        out_specs=pl.BlockSpec((1, S, Cout), lambda i: (i, 0, 0)),
        out_shape=jax.ShapeDtypeStruct((B, S, Cout), jnp.float32),
    )(feats, nxe, *wbs)


def _sa_stage(xyz_bcn, tab_bcn, npoint, radius, nsample, layers):
    B, _, N = xyz_bcn.shape
    nxyz_rows = _fps(xyz_bcn, npoint)
    D = _sa_dist(xyz_bcn, nxyz_rows)
    feats = _select_gather_jnp(D, radius ** 2, nsample, tab_bcn)
    SK = npoint * nsample
    Cf = tab_bcn.shape[1]
    feats = feats.reshape(B, SK, Cf)
    return nxyz_rows, _sa_mlp(feats, nxyz_rows, layers, nsample)


def _layer_args(p, eps=1e-5):
    return [p['W'].T, p['b'][None, :], p['mean'][None, :],
            jnp.sqrt(p['var'] + eps)[None, :], p['gamma'][None, :],
            p['beta'][None, :]]


def _layer_fwd(X, refs):
    W, b, mean, sq, gamma, beta = (r[...] for r in refs)
    y = jax.lax.dot_general(X, W, (((1,), (0,)), ((), ())),
                            preferred_element_type=jnp.float32) + b
    y = (y - mean) / sq * gamma + beta
    return jnp.maximum(y, 0.0)


def _lexlt(da, ia, db, ib):
    return (da < db) | ((da == db) & (ia < ib))


def _insert3(state, d, i):
    m1, i1, m2, i2, m3, i3 = state
    c1 = _lexlt(d, i, m1, i1)
    c2 = (~c1) & _lexlt(d, i, m2, i2)
    c3 = (~c1) & (~c2) & _lexlt(d, i, m3, i3)
    nm3 = jnp.where(c1 | c2, m2, jnp.where(c3, d, m3))
    ni3 = jnp.where(c1 | c2, i2, jnp.where(c3, i, i3))
    nm2 = jnp.where(c1, m1, jnp.where(c2, d, m2))
    ni2 = jnp.where(c1, i1, jnp.where(c2, i, i2))
    nm1 = jnp.where(c1, d, m1)
    ni1 = jnp.where(c1, i, i1)
    return nm1, ni1, nm2, ni2, nm3, ni3


def _top3_weights(qx, cxr, Nc, Nq):
    qn = jnp.sum(qx * qx, axis=0, keepdims=True)
    cn = jnp.sum(cxr * cxr, axis=1, keepdims=True)
    dot = jax.lax.dot_general(cxr, qx, (((1,), (0,)), ((), ())),
                              preferred_element_type=jnp.float32)
    Dt = (qn + cn) - 2.0 * dot

    z8 = Dt[0:8, :] * 0.0
    rio8 = jax.lax.broadcasted_iota(jnp.int32, (8, Nq), 0).astype(jnp.float32) + z8
    m1 = z8 + 1e30; m2 = z8 + 1e30; m3 = z8 + 1e30
    i1 = z8 + Nc; i2 = z8 + Nc; i3 = z8 + Nc
    for k in range(Nc // 8):
        d = Dt[k * 8:(k + 1) * 8, :]
        iv = rio8 + float(k * 8)
        c1 = d < m1
        c2 = (~c1) & (d < m2)
        c3 = (~c1) & (~c2) & (d < m3)
        m3 = jnp.where(c1 | c2, m2, jnp.where(c3, d, m3))
        i3 = jnp.where(c1 | c2, i2, jnp.where(c3, iv, i3))
        m2 = jnp.where(c1, m1, jnp.where(c2, d, m2))
        i2 = jnp.where(c1, i1, jnp.where(c2, iv, i2))
        m1 = jnp.where(c1, d, m1)
        i1 = jnp.where(c1, iv, i1)

    for half in (4, 2, 1):
        st = (m1[:half], i1[:half], m2[:half], i2[:half], m3[:half], i3[:half])
        for dd, ii in ((m1[half:2 * half], i1[half:2 * half]),
                       (m2[half:2 * half], i2[half:2 * half]),
                       (m3[half:2 * half], i3[half:2 * half])):
            st = _insert3(st, dd, ii)
        m1, i1, m2, i2, m3, i3 = st

    r1 = 1.0 / (m1 + 1e-8)
    r2 = 1.0 / (m2 + 1e-8)
    r3 = 1.0 / (m3 + 1e-8)
    norm = r1 + r2 + r3
    w1 = r1 / norm; w2 = r2 / norm; w3 = r3 / norm
    return Dt, i1, i2, i3, w1, w2, w3


def _fp_stage_kernel(nlayers, has_p1, has_conv, Nc, Nq, *refs):
    qx_ref, cx_ref, p2_ref = refs[0], refs[1], refs[2]
    pos = 3
    if has_p1:
        p1_ref = refs[pos]; pos += 1
    wb = refs[pos:pos + 6 * nlayers]; pos += 6 * nlayers
    if has_conv:
        cw_ref, cb_ref = refs[pos], refs[pos + 1]; pos += 2
    out_ref = refs[pos]

    qx = qx_ref[0]
    cxr = cx_ref[0]
    Dt, i1, i2, i3, w1, w2, w3 = _top3_weights(qx, cxr, Nc, Nq)

    rio = jax.lax.broadcasted_iota(jnp.int32, (Nc, Nq), 0).astype(jnp.float32) + Dt * 0.0
    Wt = (jnp.where(rio == i1, w1, 0.0) + jnp.where(rio == i2, w2, 0.0)
          + jnp.where(rio == i3, w3, 0.0))
    P = p2_ref[0]
    X = jax.lax.dot_general(Wt, P, (((0,), (0,)), ((), ())),
                            precision=jax.lax.Precision.HIGHEST,
                            preferred_element_type=jnp.float32)
    if has_p1:
        X = jnp.concatenate([p1_ref[0], X], axis=1)
    for l in range(nlayers):
        X = _layer_fwd(X, wb[6 * l:6 * l + 6])
    if has_conv:
        X = jax.lax.dot_general(X, cw_ref[...], (((1,), (0,)), ((), ())),
                                preferred_element_type=jnp.float32) + cb_ref[...]
    out_ref[0] = X


def _fp_stage(q_xyz, c_xyz_rows, p2_rows, p1_rows, layers, conv=None):
    B, _, Nq = q_xyz.shape
    Nc = c_xyz_rows.shape[1]
    wbs = []
    for p in layers:
        wbs += _layer_args(p)
    args = [q_xyz, c_xyz_rows, p2_rows]
    if p1_rows is not None:
        args.append(p1_rows)
    args += wbs
    if conv is not None:
        args += [conv['W'].T, conv['b'][None, :]]
        Cout = 1
    else:
        Cout = wbs[-6].shape[1]
    specs = []
    for a in args:
        if a.ndim == 3:
            specs.append(pl.BlockSpec((1,) + a.shape[1:],
                                      lambda i: (i, 0, 0)))
        else:
            specs.append(pl.BlockSpec(a.shape, lambda i: (0,) * a.ndim))
    return pl.pallas_call(
        functools.partial(_fp_stage_kernel, len(layers), p1_rows is not None,
                          conv is not None, Nc, Nq),
        grid=(B,),
        in_specs=specs,
        out_specs=pl.BlockSpec((1, Nq, Cout), lambda i: (i, 0, 0)),
        out_shape=jax.ShapeDtypeStruct((B, Nq, Cout), jnp.float32),
    )(*args)


def _fp_sel_kernel(Nc, Nq, qx_ref, cx_ref, out_ref):
    qx = qx_ref[0]
    cxr = cx_ref[0]
    _, i1, i2, i3, w1, w2, w3 = _top3_weights(qx, cxr, Nc, Nq)
    out_ref[0] = jnp.concatenate([i1, i2, i3, w1, w2, w3], axis=0)


def _fp_sel_debug(q_xyz, c_xyz_rows):
    B, _, Nq = q_xyz.shape
    Nc = c_xyz_rows.shape[1]
    out = pl.pallas_call(
        functools.partial(_fp_sel_kernel, Nc, Nq),
        grid=(B,),
        in_specs=[pl.BlockSpec((1, 3, Nq), lambda i: (i, 0, 0)),
                  pl.BlockSpec((1, Nc, 3), lambda i: (i, 0, 0))],
        out_specs=pl.BlockSpec((1, 6, Nq), lambda i: (i, 0, 0)),
        out_shape=jax.ShapeDtypeStruct((B, 6, Nq), jnp.float32),
    )(q_xyz, c_xyz_rows)
    return jnp.transpose(out, (0, 2, 1))


def _final_conv_kernel(x_ref, w_ref, b_ref, o_ref):
    o_ref[0] = jax.lax.dot_general(
        w_ref[...], x_ref[0], (((1,), (0,)), ((), ())),
        preferred_element_type=jnp.float32) + b_ref[...]


def _final_conv(x, W, b):
    B, C, N = x.shape
    return pl.pallas_call(
        _final_conv_kernel,
        grid=(B,),
        in_specs=[
            pl.BlockSpec((1, C, N), lambda i: (i, 0, 0)),
            pl.BlockSpec((1, C), lambda i: (0, 0)),
            pl.BlockSpec((1, 1), lambda i: (0, 0)),
        ],
        out_specs=pl.BlockSpec((1, 1, N), lambda i: (i, 0, 0)),
        out_shape=jax.ShapeDtypeStruct((B, 1, N), jnp.float32),
    )(x, W, b.reshape(1, 1))


def kernel(xyz, params):
    l0_xyz = xyz[:, :3, :]
    nx1_rows, l1_rows = _sa_stage(l0_xyz, xyz, 256, 0.2, 32, params['sa1'])
    nx1_bcn = jnp.transpose(nx1_rows, (0, 2, 1))
    tab2 = jnp.concatenate([nx1_bcn, jnp.transpose(l1_rows, (0, 2, 1))],
                           axis=1)
    nx2_rows, l2_rows = _sa_stage(nx1_bcn, tab2, 64, 0.4, 32, params['sa2'])
    fp2_rows = _fp_stage(nx1_bcn, nx2_rows, l2_rows, l1_rows, params['fp2'])
    out_rows = _fp_stage(l0_xyz, nx1_rows, fp2_rows, None, params['fp1'],
                         conv=params['conv1'])
    return jnp.transpose(out_rows, (0, 2, 1))


# speedup vs baseline: 1.0012x; 1.0012x over previous
import functools

import jax
import jax.numpy as jnp
import numpy as np
from jax import lax
from jax.experimental import pallas as pl
from jax.experimental.pallas import tpu as pltpu
from jax.experimental.pallas import tpu_sc as plsc


def _square_distance(src, dst):
    return jnp.sum(src ** 2, -1)[:, :, None] + jnp.sum(dst ** 2, -1)[:, None, :] - 2.0 * jnp.einsum('bnc,bmc->bnm', src, dst)


def _index_points(points, idx):
    B = points.shape[0]
    batch = jnp.arange(B).reshape((B,) + (1,) * (idx.ndim - 1))
    return points[batch, idx]


def _fps_kernel(npoint, xyz_ref, cent_ref):
    B, _, N = xyz_ref.shape
    x = xyz_ref[:, 0, :]
    y = xyz_ref[:, 1, :]
    z = xyz_ref[:, 2, :]
    zn = (x * 0.0).astype(jnp.int32)
    iota_n = jax.lax.broadcasted_iota(jnp.int32, (B, N), 1) + zn

    def body(i, state):
        distance, farthest = state
        sel = iota_n == farthest
        cx = jnp.sum(jnp.where(sel, x, 0.0), axis=1, keepdims=True)
        cy = jnp.sum(jnp.where(sel, y, 0.0), axis=1, keepdims=True)
        cz = jnp.sum(jnp.where(sel, z, 0.0), axis=1, keepdims=True)
        row = jnp.concatenate([farthest.astype(jnp.float32), cx, cy, cz],
                              axis=1)
        cent_ref[pl.ds(i, 1)] = row.reshape(1, B, 4)
        d = (x - cx) ** 2 + (y - cy) ** 2 + (z - cz) ** 2
        distance = jnp.minimum(distance, d)
        maxv = jnp.max(distance, axis=1, keepdims=True)
        farthest = jnp.min(jnp.where(distance == maxv, iota_n, N), axis=1,
                           keepdims=True)
        return distance, farthest

    distance = x * 0.0 + 1e10
    farthest = jnp.min(iota_n, axis=1, keepdims=True) * 0
    jax.lax.fori_loop(0, npoint, body, (distance, farthest))


def _fps(xyz_bcn, npoint):
    B, _, N = xyz_bcn.shape
    cent = pl.pallas_call(
        functools.partial(_fps_kernel, npoint),
        out_shape=jax.ShapeDtypeStruct((npoint, B, 4), jnp.float32),
    )(xyz_bcn)
    return jnp.transpose(cent[:, :, 1:4], (1, 0, 2))


def _query_ball(radius, nsample, xyz, new_xyz):
    B, N, C = xyz.shape
    S = new_xyz.shape[1]
    sqrdists = _square_distance(new_xyz, xyz)
    group_idx = jnp.broadcast_to(jnp.arange(N, dtype=jnp.int32), (B, S, N))
    group_idx = jnp.where(sqrdists > radius ** 2, jnp.int32(N), group_idx)
    group_idx = jnp.sort(group_idx, axis=-1)[:, :, :nsample]
    group_first = jnp.broadcast_to(group_idx[:, :, :1], group_idx.shape)
    group_idx = jnp.where(group_idx == N, group_first, group_idx)
    return jnp.clip(group_idx, 0, N - 1)


def _conv_bn_relu(x, p, eps=1e-5):
    shape = (1, -1) + (1,) * (x.ndim - 2)
    y = jnp.einsum('oi,bi...->bo...', p['W'], x) + p['b'].reshape(shape)
    y = (y - p['mean'].reshape(shape)) / jnp.sqrt(p['var'].reshape(shape) + eps) * p['gamma'].reshape(shape) + p['beta'].reshape(shape)
    return jax.nn.relu(y)


def _set_abstraction(xyz, points, npoint, radius, nsample, layers):
    xyz_t = jnp.transpose(xyz, (0, 2, 1))
    points_t = jnp.transpose(points, (0, 2, 1)) if points is not None else None
    fps_idx = _fps(xyz_t, npoint)
    new_xyz = _index_points(xyz_t, fps_idx)
    idx = _query_ball(radius, nsample, xyz_t, new_xyz)
    grouped_xyz = _index_points(xyz_t, idx) - new_xyz[:, :, None, :]
    if points_t is not None:
        grouped_points = _index_points(points_t, idx)
        new_points = jnp.concatenate([grouped_xyz, grouped_points], axis=-1)
    else:
        new_points = grouped_xyz
    new_points = jnp.transpose(new_points, (0, 3, 2, 1))
    for p in layers:
        new_points = _conv_bn_relu(new_points, p)
    new_points = jnp.max(new_points, axis=2)
    return jnp.transpose(new_xyz, (0, 2, 1)), new_points


def _feature_propagation(xyz1, xyz2, points1, points2, layers):
    xyz1_t = jnp.transpose(xyz1, (0, 2, 1))
    xyz2_t = jnp.transpose(xyz2, (0, 2, 1))
    points2_t = jnp.transpose(points2, (0, 2, 1))
    B, N, _ = xyz1_t.shape
    dists = _square_distance(xyz1_t, xyz2_t)
    idx = jnp.argsort(dists, axis=-1)[:, :, :3]
    d = jnp.take_along_axis(dists, idx, axis=-1)
    dist_recip = 1.0 / (d + 1e-8)
    norm = jnp.sum(dist_recip, axis=2, keepdims=True)
    weight = dist_recip / norm
    interpolated = jnp.sum(_index_points(points2_t, idx) * weight[..., None], axis=2)
    if points1 is not None:
        points1_t = jnp.transpose(points1, (0, 2, 1))
        new_points = jnp.concatenate([points1_t, interpolated], axis=-1)
    else:
        new_points = interpolated
    new_points = jnp.transpose(new_points, (0, 2, 1))
    for p in layers:
        new_points = _conv_bn_relu(new_points, p)
    return new_points


def _sa_dist_kernel(x_ref, nx_ref, d_ref):
    qx = x_ref[0]
    nxr = nx_ref[0]
    pn = jnp.sum(qx * qx, axis=0, keepdims=True)
    sn = jnp.sum(nxr * nxr, axis=1, keepdims=True)
    dot = jax.lax.dot_general(nxr, qx, (((1,), (0,)), ((), ())),
                              preferred_element_type=jnp.float32)
    d_ref[0] = (sn + pn) - 2.0 * dot


def _sa_dist(xyz_bcn, nxyz_rows):
    B, _, N = xyz_bcn.shape
    S = nxyz_rows.shape[1]
    return pl.pallas_call(
        _sa_dist_kernel,
        grid=(B,),
        in_specs=[pl.BlockSpec((1, 3, N), lambda i: (i, 0, 0)),
                  pl.BlockSpec((1, S, 3), lambda i: (i, 0, 0))],
        out_specs=pl.BlockSpec((1, S, N), lambda i: (i, 0, 0)),
        out_shape=jax.ShapeDtypeStruct((B, S, N), jnp.float32),
    )(xyz_bcn, nxyz_rows)


def _select_gather_jnp(D, r2, nsample, tab):
    B, S, N = D.shape
    gi = jnp.broadcast_to(jnp.arange(N, dtype=jnp.int32), (B, S, N))
    gi = jnp.where(D > r2, jnp.int32(N), gi)
    gi = jnp.sort(gi, axis=-1)[:, :, :nsample]
    first = jnp.broadcast_to(gi[:, :, :1], gi.shape)
    gi = jnp.where(gi == N, first, gi)
    gi = jnp.clip(gi, 0, N - 1)
    bidx = jnp.arange(B).reshape(B, 1, 1)
    g = tab[bidx, :, gi]
    return g.reshape(B, S, -1)


def _sa_mlp_kernel(nlayers, S, ns, x_ref, nx_ref, *rest):
    wb = rest[:6 * nlayers]
    out_ref = rest[6 * nlayers]
    F = x_ref[0]
    NX = nx_ref[0]
    rel = F[:, 0:3] - NX
    pts = F if F.shape[1] == 3 else F[:, 3:]
    X = jnp.concatenate([rel, pts], axis=1)
    for l in range(nlayers):
        X = _layer_fwd(X, wb[6 * l:6 * l + 6])
    C = X.shape[1]
    out_ref[0] = jnp.max(X.reshape(S, ns, C), axis=1)


def _sa_mlp(feats, nxyz_rows, layers, nsample):
    B, SK, Cf = feats.shape
    S = SK // nsample
    nxe = jnp.repeat(nxyz_rows, nsample, axis=1)
    wbs = []
    for p in layers:
        wbs += _layer_args(p)
    Cout = wbs[-6].shape[1]
    specs = [pl.BlockSpec((1, SK, Cf), lambda i: (i, 0, 0)),
             pl.BlockSpec((1, SK, 3), lambda i: (i, 0, 0))]
    for a in wbs:
        specs.append(pl.BlockSpec(a.shape, lambda i: (0,) * a.ndim))
    return pl.pallas_call(
        functools.partial(_sa_mlp_kernel, len(layers), S, nsample),
        grid=(B,),
        in_specs=specs,
        out_specs=pl.BlockSpec((1, S, Cout), lambda i: (i, 0, 0)),
        out_shape=jax.ShapeDtypeStruct((B, S, Cout), jnp.float32),
    )(feats, nxe, *wbs)


def _sa_stage(xyz_bcn, tab_bcn, npoint, radius, nsample, layers):
    B, _, N = xyz_bcn.shape
    nxyz_rows = _fps(xyz_bcn, npoint)
    D = _sa_dist(xyz_bcn, nxyz_rows)
    feats = _select_gather_jnp(D, radius ** 2, nsample, tab_bcn)
    SK = npoint * nsample
    Cf = tab_bcn.shape[1]
    feats = feats.reshape(B, SK, Cf)
    return nxyz_rows, _sa_mlp(feats, nxyz_rows, layers, nsample)


def _layer_args(p, eps=1e-5):
    return [p['W'].T, p['b'][None, :], p['mean'][None, :],
            jnp.sqrt(p['var'] + eps)[None, :], p['gamma'][None, :],
            p['beta'][None, :]]


def _layer_fwd(X, refs):
    W, b, mean, sq, gamma, beta = (r[...] for r in refs)
    y = jax.lax.dot_general(X, W, (((1,), (0,)), ((), ())),
                            preferred_element_type=jnp.float32) + b
    y = (y - mean) / sq * gamma + beta
    return jnp.maximum(y, 0.0)


def _lexlt(da, ia, db, ib):
    return (da < db) | ((da == db) & (ia < ib))


def _insert3(state, d, i):
    m1, i1, m2, i2, m3, i3 = state
    c1 = _lexlt(d, i, m1, i1)
    c2 = (~c1) & _lexlt(d, i, m2, i2)
    c3 = (~c1) & (~c2) & _lexlt(d, i, m3, i3)
    nm3 = jnp.where(c1 | c2, m2, jnp.where(c3, d, m3))
    ni3 = jnp.where(c1 | c2, i2, jnp.where(c3, i, i3))
    nm2 = jnp.where(c1, m1, jnp.where(c2, d, m2))
    ni2 = jnp.where(c1, i1, jnp.where(c2, i, i2))
    nm1 = jnp.where(c1, d, m1)
    ni1 = jnp.where(c1, i, i1)
    return nm1, ni1, nm2, ni2, nm3, ni3


def _top3_weights(Dt, Nc, Nq):

    z8 = Dt[0:8, :] * 0.0
    rio8 = jax.lax.broadcasted_iota(jnp.int32, (8, Nq), 0).astype(jnp.float32) + z8
    m1 = z8 + 1e30; m2 = z8 + 1e30; m3 = z8 + 1e30
    i1 = z8 + Nc; i2 = z8 + Nc; i3 = z8 + Nc
    for k in range(Nc // 8):
        d = Dt[k * 8:(k + 1) * 8, :]
        iv = rio8 + float(k * 8)
        c1 = d < m1
        c2 = (~c1) & (d < m2)
        c3 = (~c1) & (~c2) & (d < m3)
        m3 = jnp.where(c1 | c2, m2, jnp.where(c3, d, m3))
        i3 = jnp.where(c1 | c2, i2, jnp.where(c3, iv, i3))
        m2 = jnp.where(c1, m1, jnp.where(c2, d, m2))
        i2 = jnp.where(c1, i1, jnp.where(c2, iv, i2))
        m1 = jnp.where(c1, d, m1)
        i1 = jnp.where(c1, iv, i1)

    for half in (4, 2, 1):
        st = (m1[:half], i1[:half], m2[:half], i2[:half], m3[:half], i3[:half])
        for dd, ii in ((m1[half:2 * half], i1[half:2 * half]),
                       (m2[half:2 * half], i2[half:2 * half]),
                       (m3[half:2 * half], i3[half:2 * half])):
            st = _insert3(st, dd, ii)
        m1, i1, m2, i2, m3, i3 = st

    r1 = 1.0 / (m1 + 1e-8)
    r2 = 1.0 / (m2 + 1e-8)
    r3 = 1.0 / (m3 + 1e-8)
    norm = r1 + r2 + r3
    w1 = r1 / norm; w2 = r2 / norm; w3 = r3 / norm
    return None, i1, i2, i3, w1, w2, w3


def _fp_stage_kernel(nlayers, has_p1, has_conv, Nc, Nq, *refs):
    dt_ref, p2_ref = refs[0], refs[1]
    pos = 2
    if has_p1:
        p1_ref = refs[pos]; pos += 1
    wb = refs[pos:pos + 6 * nlayers]; pos += 6 * nlayers
    if has_conv:
        cw_ref, cb_ref = refs[pos], refs[pos + 1]; pos += 2
    out_ref = refs[pos]

    Dt = jnp.transpose(dt_ref[0])
    _, i1, i2, i3, w1, w2, w3 = _top3_weights(Dt, Nc, Nq)

    rio = jax.lax.broadcasted_iota(jnp.int32, (Nc, Nq), 0).astype(jnp.float32) + Dt * 0.0
    Wt = (jnp.where(rio == i1, w1, 0.0) + jnp.where(rio == i2, w2, 0.0)
          + jnp.where(rio == i3, w3, 0.0))
    P = p2_ref[0]
    X = jax.lax.dot_general(Wt, P, (((0,), (0,)), ((), ())),
                            precision=jax.lax.Precision.HIGHEST,
                            preferred_element_type=jnp.float32)
    if has_p1:
        X = jnp.concatenate([p1_ref[0], X], axis=1)
    for l in range(nlayers):
        X = _layer_fwd(X, wb[6 * l:6 * l + 6])
    if has_conv:
        X = jax.lax.dot_general(X, cw_ref[...], (((1,), (0,)), ((), ())),
                                preferred_element_type=jnp.float32) + cb_ref[...]
    out_ref[0] = X


def _fp_stage(D, p2_rows, p1_rows, layers, conv=None):
    B, Nq, Nc = D.shape
    wbs = []
    for p in layers:
        wbs += _layer_args(p)
    args = [D, p2_rows]
    if p1_rows is not None:
        args.append(p1_rows)
    args += wbs
    if conv is not None:
        args += [conv['W'].T, conv['b'][None, :]]
        Cout = 1
    else:
        Cout = wbs[-6].shape[1]
    specs = []
    for a in args:
        if a.ndim == 3:
            specs.append(pl.BlockSpec((1,) + a.shape[1:],
                                      lambda i: (i, 0, 0)))
        else:
            specs.append(pl.BlockSpec(a.shape, lambda i: (0,) * a.ndim))
    return pl.pallas_call(
        functools.partial(_fp_stage_kernel, len(layers), p1_rows is not None,
                          conv is not None, Nc, Nq),
        grid=(B,),
        in_specs=specs,
        out_specs=pl.BlockSpec((1, Nq, Cout), lambda i: (i, 0, 0)),
        out_shape=jax.ShapeDtypeStruct((B, Nq, Cout), jnp.float32),
    )(*args)


def _fp_sel_kernel(Nc, Nq, qx_ref, cx_ref, out_ref):
    qx = qx_ref[0]
    cxr = cx_ref[0]
    _, i1, i2, i3, w1, w2, w3 = _top3_weights(qx, cxr, Nc, Nq)
    out_ref[0] = jnp.concatenate([i1, i2, i3, w1, w2, w3], axis=0)


def _fp_sel_debug(q_xyz, c_xyz_rows):
    B, _, Nq = q_xyz.shape
    Nc = c_xyz_rows.shape[1]
    out = pl.pallas_call(
        functools.partial(_fp_sel_kernel, Nc, Nq),
        grid=(B,),
        in_specs=[pl.BlockSpec((1, 3, Nq), lambda i: (i, 0, 0)),
                  pl.BlockSpec((1, Nc, 3), lambda i: (i, 0, 0))],
        out_specs=pl.BlockSpec((1, 6, Nq), lambda i: (i, 0, 0)),
        out_shape=jax.ShapeDtypeStruct((B, 6, Nq), jnp.float32),
    )(q_xyz, c_xyz_rows)
    return jnp.transpose(out, (0, 2, 1))


def _final_conv_kernel(x_ref, w_ref, b_ref, o_ref):
    o_ref[0] = jax.lax.dot_general(
        w_ref[...], x_ref[0], (((1,), (0,)), ((), ())),
        preferred_element_type=jnp.float32) + b_ref[...]


def _final_conv(x, W, b):
    B, C, N = x.shape
    return pl.pallas_call(
        _final_conv_kernel,
        grid=(B,),
        in_specs=[
            pl.BlockSpec((1, C, N), lambda i: (i, 0, 0)),
            pl.BlockSpec((1, C), lambda i: (0, 0)),
            pl.BlockSpec((1, 1), lambda i: (0, 0)),
        ],
        out_specs=pl.BlockSpec((1, 1, N), lambda i: (i, 0, 0)),
        out_shape=jax.ShapeDtypeStruct((B, 1, N), jnp.float32),
    )(x, W, b.reshape(1, 1))


def kernel(xyz, params):
    l0_xyz = xyz[:, :3, :]
    nx1_rows, l1_rows = _sa_stage(l0_xyz, xyz, 256, 0.2, 32, params['sa1'])
    nx1_bcn = jnp.transpose(nx1_rows, (0, 2, 1))
    tab2 = jnp.concatenate([nx1_bcn, jnp.transpose(l1_rows, (0, 2, 1))],
                           axis=1)
    nx2_rows, l2_rows = _sa_stage(nx1_bcn, tab2, 64, 0.4, 32, params['sa2'])
    D2 = _square_distance(nx1_rows, nx2_rows)
    fp2_rows = _fp_stage(D2, l2_rows, l1_rows, params['fp2'])
    D1 = _square_distance(jnp.transpose(l0_xyz, (0, 2, 1)), nx1_rows)
    out_rows = _fp_stage(D1, fp2_rows, None, params['fp1'],
                         conv=params['conv1'])
    return jnp.transpose(out_rows, (0, 2, 1))
